# baseline (speedup 1.0000x reference)
import functools

import jax
import jax.numpy as jnp
from jax import lax
from jax.experimental import pallas as pl
from jax.experimental.pallas import tpu as pltpu
from jax.experimental.pallas import tpu_sc as plsc

_N = 5000
_CONF_THRE = 0.1
_NMS_THRE = 0.45
_BL = 1024
_NB = 5
_NP = _NB * _BL
_CH = 1
_NCOL = 7
_SLC = _NP // 32


_FLAT = _NCOL * _NP
_PER_W = _FLAT // 32
_GCH = 80


def _gather_body(predt_hbm, order7_hbm, out_hbm, idx_v, ob_v, sem):
    wid = lax.axis_index("s") * 2 + lax.axis_index("c")
    base = wid * _PER_W
    pltpu.sync_copy(order7_hbm.at[pl.ds(base, _PER_W)], idx_v)
    copies = []
    for k in range(_PER_W // _GCH):
        copies.append(pltpu.async_copy(
            predt_hbm.at[idx_v.at[pl.ds(k * _GCH, _GCH)]],
            ob_v.at[pl.ds(k * _GCH, _GCH)], sem))
    for cp in copies:
        cp.wait()
    pltpu.sync_copy(ob_v, out_hbm.at[pl.ds(base, _PER_W)])


def _sc_gather(predt_p, order7):
    mesh = plsc.VectorSubcoreMesh(core_axis_name="c", subcore_axis_name="s")
    f = functools.partial(
        pl.kernel,
        mesh=mesh,
        out_type=jax.ShapeDtypeStruct((_FLAT,), jnp.float32),
        scratch_types=[
            pltpu.VMEM((_PER_W,), jnp.int32),
            pltpu.VMEM((_PER_W,), jnp.float32),
            pltpu.SemaphoreType.DMA,
        ],
    )(_gather_body)
    return f(predt_p, order7)


def _nms_body(d_ref, o_ref, sx1, sy1, sx2, sy2, sar, sva, skp):
    x1 = d_ref[0]
    y1 = d_ref[1]
    x2 = d_ref[2]
    y2 = d_ref[3]
    score = d_ref[4] * d_ref[5]
    cls = d_ref[6]
    valid = score >= _CONF_THRE
    max_coord = jnp.max(jnp.where(valid, jnp.maximum(x2, y2), 0.0)) + 1.0
    off = cls * max_coord
    sx1[...] = x1 + off
    sy1[...] = y1 + off
    sx2[...] = x2 + off
    sy2[...] = y2 + off
    sar[...] = (x2 - x1) * (y2 - y1)
    sva[...] = valid.astype(jnp.float32)
    skp[...] = jnp.zeros((_NB, _BL), jnp.float32)

    tri = (jax.lax.broadcasted_iota(jnp.int32, (_BL, _BL), 0)
           < jax.lax.broadcasted_iota(jnp.int32, (_BL, _BL), 1)
           ).astype(jnp.float32)

    def block_step(b, carry):
        @pl.when(jnp.any(sva[pl.ds(b, 1), :] > 0.0))
        def _():
            rx1 = sx1[pl.ds(b, 1), :]
            ry1 = sy1[pl.ds(b, 1), :]
            rx2 = sx2[pl.ds(b, 1), :]
            ry2 = sy2[pl.ds(b, 1), :]
            rar = sar[pl.ds(b, 1), :]
            cx1 = jnp.transpose(rx1)
            cy1 = jnp.transpose(ry1)
            cx2 = jnp.transpose(rx2)
            cy2 = jnp.transpose(ry2)
            car = jnp.transpose(rar)

            def cross(ci, acc):
                base = ci * _CH
                for u in range(_CH):
                    a = base + u
                    ax1 = sx1[pl.ds(a, 1), :]
                    ay1 = sy1[pl.ds(a, 1), :]
                    ax2 = sx2[pl.ds(a, 1), :]
                    ay2 = sy2[pl.ds(a, 1), :]
                    aar = sar[pl.ds(a, 1), :]
                    ka = skp[pl.ds(a, 1), :]
                    iw = jnp.maximum(
                        jnp.minimum(cx2, ax2) - jnp.maximum(cx1, ax1), 0.0)
                    ih = jnp.maximum(
                        jnp.minimum(cy2, ay2) - jnp.maximum(cy1, ay1), 0.0)
                    inter = iw * ih
                    union = jnp.maximum(car + aar - inter, 1e-9)
                    hit = jnp.where(inter / union > _NMS_THRE, ka, 0.0)
                    acc = jnp.maximum(acc, hit)
                return acc

            nch = (b + _CH - 1) // _CH
            acc = jax.lax.fori_loop(
                0, nch, cross, jnp.zeros((_BL, _BL), jnp.float32))
            supp = jnp.max(acc, axis=1, keepdims=True) > 0.0
            veff = jnp.where(jnp.transpose(supp), 0.0, sva[pl.ds(b, 1), :])

            iw = jnp.maximum(jnp.minimum(cx2, rx2) - jnp.maximum(cx1, rx1),
                             0.0)
            ih = jnp.maximum(jnp.minimum(cy2, ry2) - jnp.maximum(cy1, ry1),
                             0.0)
            inter = iw * ih
            union = jnp.maximum(car + rar - inter, 1e-9)
            mbb = jnp.where(inter / union > _NMS_THRE, 1.0, 0.0) * tri

            def w_cond(c):
                return jnp.logical_not(c[1])

            def w_body(c):
                k = c[0]
                s = jax.lax.dot_general(k, mbb, (((1,), (0,)), ((), ())),
                                        preferred_element_type=jnp.float32)
                nk = jnp.where(s > 0.0, 0.0, veff)
                return (nk, jnp.all(nk == k))

            kfin, _ = jax.lax.while_loop(w_cond, w_body,
                                         (veff, jnp.array(False)))
            skp[pl.ds(b, 1), :] = kfin

        return carry

    jax.lax.fori_loop(0, _NB, block_step, 0)
    o_ref[...] = d_ref[...] * skp[...][None]


def kernel(pred):
    scores = pred[:, 4] * pred[:, 5]
    conf_mask = scores >= _CONF_THRE
    order = jnp.argsort(-jnp.where(conf_mask, scores, -jnp.inf))
    order_p = jnp.concatenate(
        [order.astype(jnp.int32),
         jnp.full(_NP - _N, _NP - 1, jnp.int32)])
    order7 = (order_p[None, :]
              + (_NP * jnp.arange(_NCOL, dtype=jnp.int32))[:, None]
              ).reshape(_NCOL * _NP)
    predt_p = jnp.concatenate(
        [pred.T, jnp.zeros((_NCOL, _NP - _N), jnp.float32)],
        axis=1).reshape(_NCOL * _NP)
    d3 = _sc_gather(predt_p, order7).reshape(_NCOL, _NB, _BL)
    out3 = pl.pallas_call(
        _nms_body,
        out_shape=jax.ShapeDtypeStruct((_NCOL, _NB, _BL), jnp.float32),
        scratch_shapes=[pltpu.VMEM((_NB, _BL), jnp.float32)] * 7,
    )(d3)
    return out3.reshape(_NCOL, _NP).T[:_N]


# speedup vs baseline: 1.0697x; 1.0697x over previous
import functools

import jax
import jax.numpy as jnp
from jax import lax
from jax.experimental import pallas as pl
from jax.experimental.pallas import tpu as pltpu
from jax.experimental.pallas import tpu_sc as plsc

_N = 5000
_CONF_THRE = 0.1
_NMS_THRE = 0.45
_BL = 512
_NB = 10
_NP = _NB * _BL
_CH = 4
_NCOL = 7
_SLC = _NP // 32


_FLAT = _NCOL * _NP
_PER_W = _FLAT // 32
_GCH = 80


def _gather_body(predt_hbm, order7_hbm, out_hbm, idx_v, ob_v, sem):
    wid = lax.axis_index("s") * 2 + lax.axis_index("c")
    base = wid * _PER_W
    pltpu.sync_copy(order7_hbm.at[pl.ds(base, _PER_W)], idx_v)
    copies = []
    for k in range(_PER_W // _GCH):
        copies.append(pltpu.async_copy(
            predt_hbm.at[idx_v.at[pl.ds(k * _GCH, _GCH)]],
            ob_v.at[pl.ds(k * _GCH, _GCH)], sem))
    for cp in copies:
        cp.wait()
    pltpu.sync_copy(ob_v, out_hbm.at[pl.ds(base, _PER_W)])


def _sc_gather(predt_p, order7):
    mesh = plsc.VectorSubcoreMesh(core_axis_name="c", subcore_axis_name="s")
    f = functools.partial(
        pl.kernel,
        mesh=mesh,
        out_type=jax.ShapeDtypeStruct((_FLAT,), jnp.float32),
        scratch_types=[
            pltpu.VMEM((_PER_W,), jnp.int32),
            pltpu.VMEM((_PER_W,), jnp.float32),
            pltpu.SemaphoreType.DMA,
        ],
    )(_gather_body)
    return f(predt_p, order7)


def _nms_body(d_ref, o_ref, sx1, sy1, sx2, sy2, sar, sva, skp):
    x1 = d_ref[0]
    y1 = d_ref[1]
    x2 = d_ref[2]
    y2 = d_ref[3]
    score = d_ref[4] * d_ref[5]
    cls = d_ref[6]
    valid = score >= _CONF_THRE
    max_coord = jnp.max(jnp.where(valid, jnp.maximum(x2, y2), 0.0)) + 1.0
    off = cls * max_coord
    sx1[...] = x1 + off
    sy1[...] = y1 + off
    sx2[...] = x2 + off
    sy2[...] = y2 + off
    sar[...] = (x2 - x1) * (y2 - y1)
    sva[...] = valid.astype(jnp.float32)
    skp[...] = jnp.zeros((_NB, _BL), jnp.float32)

    tri = (jax.lax.broadcasted_iota(jnp.int32, (_BL, _BL), 0)
           < jax.lax.broadcasted_iota(jnp.int32, (_BL, _BL), 1)
           ).astype(jnp.float32)

    def block_step(b, carry):
        @pl.when(jnp.any(sva[pl.ds(b, 1), :] > 0.0))
        def _():
            rx1 = sx1[pl.ds(b, 1), :]
            ry1 = sy1[pl.ds(b, 1), :]
            rx2 = sx2[pl.ds(b, 1), :]
            ry2 = sy2[pl.ds(b, 1), :]
            rar = sar[pl.ds(b, 1), :]
            cx1 = jnp.transpose(rx1)
            cy1 = jnp.transpose(ry1)
            cx2 = jnp.transpose(rx2)
            cy2 = jnp.transpose(ry2)
            car = jnp.transpose(rar)

            def cross(ci, acc):
                base = ci * _CH
                for u in range(_CH):
                    a = base + u
                    ax1 = sx1[pl.ds(a, 1), :]
                    ay1 = sy1[pl.ds(a, 1), :]
                    ax2 = sx2[pl.ds(a, 1), :]
                    ay2 = sy2[pl.ds(a, 1), :]
                    aar = sar[pl.ds(a, 1), :]
                    ka = skp[pl.ds(a, 1), :]
                    iw = jnp.maximum(
                        jnp.minimum(cx2, ax2) - jnp.maximum(cx1, ax1), 0.0)
                    ih = jnp.maximum(
                        jnp.minimum(cy2, ay2) - jnp.maximum(cy1, ay1), 0.0)
                    inter = iw * ih
                    union = jnp.maximum(car + aar - inter, 1e-9)
                    hit = jnp.where(inter / union > _NMS_THRE, ka, 0.0)
                    acc = jnp.maximum(acc, hit)
                return acc

            nch = (b + _CH - 1) // _CH
            acc = jax.lax.fori_loop(
                0, nch, cross, jnp.zeros((_BL, _BL), jnp.float32))
            supp = jnp.max(acc, axis=1, keepdims=True) > 0.0
            veff = jnp.where(jnp.transpose(supp), 0.0, sva[pl.ds(b, 1), :])

            iw = jnp.maximum(jnp.minimum(cx2, rx2) - jnp.maximum(cx1, rx1),
                             0.0)
            ih = jnp.maximum(jnp.minimum(cy2, ry2) - jnp.maximum(cy1, ry1),
                             0.0)
            inter = iw * ih
            union = jnp.maximum(car + rar - inter, 1e-9)
            mbb = jnp.where(inter / union > _NMS_THRE, 1.0, 0.0) * tri

            def w_cond(c):
                return jnp.logical_not(c[1])

            def w_body(c):
                k = c[0]
                s = jax.lax.dot_general(k, mbb, (((1,), (0,)), ((), ())),
                                        preferred_element_type=jnp.float32)
                nk = jnp.where(s > 0.0, 0.0, veff)
                return (nk, jnp.all(nk == k))

            kfin, _ = jax.lax.while_loop(w_cond, w_body,
                                         (veff, jnp.array(False)))
            skp[pl.ds(b, 1), :] = kfin

        return carry

    jax.lax.fori_loop(0, _NB, block_step, 0)
    o_ref[...] = d_ref[...] * skp[...][None]


def kernel(pred):
    scores = pred[:, 4] * pred[:, 5]
    conf_mask = scores >= _CONF_THRE
    order = jnp.argsort(-jnp.where(conf_mask, scores, -jnp.inf))
    order_p = jnp.concatenate(
        [order.astype(jnp.int32),
         jnp.full(_NP - _N, _NP - 1, jnp.int32)])
    order7 = (order_p[None, :]
              + (_NP * jnp.arange(_NCOL, dtype=jnp.int32))[:, None]
              ).reshape(_NCOL * _NP)
    predt_p = jnp.concatenate(
        [pred.T, jnp.zeros((_NCOL, _NP - _N), jnp.float32)],
        axis=1).reshape(_NCOL * _NP)
    d3 = _sc_gather(predt_p, order7).reshape(_NCOL, _NB, _BL)
    out3 = pl.pallas_call(
        _nms_body,
        out_shape=jax.ShapeDtypeStruct((_NCOL, _NB, _BL), jnp.float32),
        scratch_shapes=[pltpu.VMEM((_NB, _BL), jnp.float32)] * 7,
    )(d3)
    return out3.reshape(_NCOL, _NP).T[:_N]


# speedup vs baseline: 1.1177x; 1.0448x over previous
import functools

import jax
import jax.numpy as jnp
from jax import lax
from jax.experimental import pallas as pl
from jax.experimental.pallas import tpu as pltpu
from jax.experimental.pallas import tpu_sc as plsc

_N = 5000
_CONF_THRE = 0.1
_NMS_THRE = 0.45
_BL = 512
_NB = 10
_NP = _NB * _BL
_CH = 2
_NCOL = 7
_SLC = _NP // 32


_FLAT = _NCOL * _NP
_PER_W = _FLAT // 32
_GCH = 80


def _gather_body(predt_hbm, order7_hbm, out_hbm, idx_v, ob_v, sem):
    wid = lax.axis_index("s") * 2 + lax.axis_index("c")
    base = wid * _PER_W
    pltpu.sync_copy(order7_hbm.at[pl.ds(base, _PER_W)], idx_v)
    copies = []
    for k in range(_PER_W // _GCH):
        copies.append(pltpu.async_copy(
            predt_hbm.at[idx_v.at[pl.ds(k * _GCH, _GCH)]],
            ob_v.at[pl.ds(k * _GCH, _GCH)], sem))
    for cp in copies:
        cp.wait()
    pltpu.sync_copy(ob_v, out_hbm.at[pl.ds(base, _PER_W)])


def _sc_gather(predt_p, order7):
    mesh = plsc.VectorSubcoreMesh(core_axis_name="c", subcore_axis_name="s")
    f = functools.partial(
        pl.kernel,
        mesh=mesh,
        out_type=jax.ShapeDtypeStruct((_FLAT,), jnp.float32),
        scratch_types=[
            pltpu.VMEM((_PER_W,), jnp.int32),
            pltpu.VMEM((_PER_W,), jnp.float32),
            pltpu.SemaphoreType.DMA,
        ],
    )(_gather_body)
    return f(predt_p, order7)


def _nms_body(d_ref, o_ref, sx1, sy1, sx2, sy2, sar, sva, skp):
    x1 = d_ref[0]
    y1 = d_ref[1]
    x2 = d_ref[2]
    y2 = d_ref[3]
    score = d_ref[4] * d_ref[5]
    cls = d_ref[6]
    valid = score >= _CONF_THRE
    max_coord = jnp.max(jnp.where(valid, jnp.maximum(x2, y2), 0.0)) + 1.0
    off = cls * max_coord
    sx1[...] = x1 + off
    sy1[...] = y1 + off
    sx2[...] = x2 + off
    sy2[...] = y2 + off
    sar[...] = (x2 - x1) * (y2 - y1)
    sva[...] = valid.astype(jnp.float32)
    skp[...] = jnp.zeros((_NB, _BL), jnp.float32)

    tri = (jax.lax.broadcasted_iota(jnp.int32, (_BL, _BL), 0)
           < jax.lax.broadcasted_iota(jnp.int32, (_BL, _BL), 1)
           ).astype(jnp.float32)

    def block_step(b, carry):
        @pl.when(jnp.any(sva[pl.ds(b, 1), :] > 0.0))
        def _():
            rx1 = sx1[pl.ds(b, 1), :]
            ry1 = sy1[pl.ds(b, 1), :]
            rx2 = sx2[pl.ds(b, 1), :]
            ry2 = sy2[pl.ds(b, 1), :]
            rar = sar[pl.ds(b, 1), :]
            cx1 = jnp.transpose(rx1)
            cy1 = jnp.transpose(ry1)
            cx2 = jnp.transpose(rx2)
            cy2 = jnp.transpose(ry2)
            car = jnp.transpose(rar)

            def cross(ci, acc):
                base = ci * _CH
                for u in range(_CH):
                    a = base + u
                    ax1 = sx1[pl.ds(a, 1), :]
                    ay1 = sy1[pl.ds(a, 1), :]
                    ax2 = sx2[pl.ds(a, 1), :]
                    ay2 = sy2[pl.ds(a, 1), :]
                    aar = sar[pl.ds(a, 1), :]
                    ka = skp[pl.ds(a, 1), :]
                    iw = jnp.maximum(
                        jnp.minimum(cx2, ax2) - jnp.maximum(cx1, ax1), 0.0)
                    ih = jnp.maximum(
                        jnp.minimum(cy2, ay2) - jnp.maximum(cy1, ay1), 0.0)
                    inter = iw * ih
                    union = jnp.maximum(car + aar - inter, 1e-9)
                    hit = jnp.where(inter / union > _NMS_THRE, ka, 0.0)
                    acc = jnp.maximum(acc, hit)
                return acc

            nch = (b + _CH - 1) // _CH
            acc = jax.lax.fori_loop(
                0, nch, cross, jnp.zeros((_BL, _BL), jnp.float32))
            supp = jnp.max(acc, axis=1, keepdims=True) > 0.0
            veff = jnp.where(jnp.transpose(supp), 0.0, sva[pl.ds(b, 1), :])

            iw = jnp.maximum(jnp.minimum(cx2, rx2) - jnp.maximum(cx1, rx1),
                             0.0)
            ih = jnp.maximum(jnp.minimum(cy2, ry2) - jnp.maximum(cy1, ry1),
                             0.0)
            inter = iw * ih
            union = jnp.maximum(car + rar - inter, 1e-9)
            mbb = jnp.where(inter / union > _NMS_THRE, 1.0, 0.0) * tri

            def w_cond(c):
                return jnp.logical_not(c[1])

            def w_body(c):
                k = c[0]
                s = jax.lax.dot_general(k, mbb, (((1,), (0,)), ((), ())),
                                        preferred_element_type=jnp.float32)
                nk = jnp.where(s > 0.0, 0.0, veff)
                return (nk, jnp.all(nk == k))

            kfin, _ = jax.lax.while_loop(w_cond, w_body,
                                         (veff, jnp.array(False)))
            skp[pl.ds(b, 1), :] = kfin

        return carry

    jax.lax.fori_loop(0, _NB, block_step, 0)
    o_ref[...] = d_ref[...] * skp[...][None]


def kernel(pred):
    scores = pred[:, 4] * pred[:, 5]
    conf_mask = scores >= _CONF_THRE
    order = jnp.argsort(-jnp.where(conf_mask, scores, -jnp.inf))
    order_p = jnp.concatenate(
        [order.astype(jnp.int32),
         jnp.full(_NP - _N, _NP - 1, jnp.int32)])
    order7 = (order_p[None, :]
              + (_NP * jnp.arange(_NCOL, dtype=jnp.int32))[:, None]
              ).reshape(_NCOL * _NP)
    predt_p = jnp.concatenate(
        [pred.T, jnp.zeros((_NCOL, _NP - _N), jnp.float32)],
        axis=1).reshape(_NCOL * _NP)
    d3 = _sc_gather(predt_p, order7).reshape(_NCOL, _NB, _BL)
    out3 = pl.pallas_call(
        _nms_body,
        out_shape=jax.ShapeDtypeStruct((_NCOL, _NB, _BL), jnp.float32),
        scratch_shapes=[pltpu.VMEM((_NB, _BL), jnp.float32)] * 7,
    )(d3)
    return out3.reshape(_NCOL, _NP).T[:_N]
